# Initial kernel scaffold; baseline (speedup 1.0000x reference)
import jax
import jax.numpy as jnp
from jax.experimental import pallas as pl


def kernel(users, items, features, edge_row, edge_col, edge_val, W1, b1, W2, b2, Wt, bt):
    raise NotImplementedError("write your pallas kernel here")


# speedup vs baseline: 5.0072x; 5.0072x over previous
import functools

import jax
import jax.numpy as jnp
from jax import lax
from jax.experimental import pallas as pl
from jax.experimental.pallas import tpu as pltpu
from jax.experimental.pallas import tpu_sc as plsc

NU = 30000
NI = 20000
NNODES = NU + NI
D = 64
NLAYER = 3
NC, NS = 2, 16
CH = 32
TCH = 800
EH_PAD = NS * TCH * CH
RPT0, RPT0_LAST = 1880, 1800
RPT1, RPT1_LAST = 1256, 1160

_MESH = plsc.VectorSubcoreMesh(
    core_axis_name="c", subcore_axis_name="s", num_cores=NC, num_subcores=NS)


def _lane_bcast(v, r):
    idx = jnp.full((16, 1), r, jnp.int32)
    return lax.gather(
        v, idx,
        lax.GatherDimensionNumbers(
            offset_dims=(), collapsed_slice_dims=(0,), start_index_map=(0,)),
        (1,), mode=lax.GatherScatterMode.PROMISE_IN_BOUNDS)


G = 4
NSLOT = 3
NG = TCH // G
NBUF = 4
ZROWS = NBUF * CH


def _spmm_body(feat, er2, ec2, ev, out, agg, gs0, gs1, ss0, ss1, isem):
    pl.run_scoped(
        functools.partial(_spmm_inner, feat, er2, ec2, ev, out, agg,
                          gs0, gs1, ss0, ss1, isem),
        colb=pltpu.VMEM((NSLOT * G, CH), jnp.int32),
        rowb=pltpu.VMEM((NSLOT * G, CH), jnp.int32),
        valb=pltpu.VMEM((NSLOT * G * CH,), jnp.float32),
        rbuf=pltpu.VMEM((ZROWS, D), jnp.float32),
    )


def _spmm_inner(feat, er2, ec2, ev, out, agg, gs0, gs1, ss0, ss1, isem,
                colb, rowb, valb, rbuf):
    c = lax.axis_index("c")
    s = lax.axis_index("s")
    gsems = (gs0, gs1)
    ssems = (ss0, ss1)

    def _z(i, carry):
        for q in range(D // 16):
            rbuf[i, pl.ds(q * 16, 16)] = jnp.zeros((16,), jnp.float32)
        return carry
    lax.fori_loop(0, ZROWS, _z, 0)
    for j in range(RPT0 // ZROWS):
        pltpu.sync_copy(rbuf, agg.at[pl.ds(s * RPT0 + j * ZROWS, ZROWS)])
    nfull = (RPT0 // ZROWS) * ZROWS
    @pl.when(s < NS - 1)
    def _():
        pltpu.sync_copy(rbuf.at[pl.ds(0, RPT0 - nfull)],
                        agg.at[pl.ds(s * RPT0 + nfull, RPT0 - nfull)])
    @pl.when(s == NS - 1)
    def _():
        pltpu.sync_copy(rbuf.at[pl.ds(0, RPT0_LAST - nfull)],
                        agg.at[pl.ds(s * RPT0 + nfull, RPT0_LAST - nfull)])
    plsc.subcore_barrier()

    w = (c * NS + s) * TCH
    offv = jnp.full((16,), c * NU, jnp.int32)

    def _stage_sync(gw, slot):
        pltpu.sync_copy(er2.at[pl.ds(gw, G)], rowb.at[pl.ds(slot * G, G)])
        pltpu.sync_copy(ec2.at[pl.ds(gw, G)], colb.at[pl.ds(slot * G, G)])
        pltpu.sync_copy(ev.at[pl.ds(gw * CH, G * CH)],
                        valb.at[pl.ds(slot * G * CH, G * CH)])

    def _stage_async(gw, slot):
        pltpu.async_copy(er2.at[pl.ds(gw, G)],
                         rowb.at[pl.ds(slot * G, G)], isem)
        pltpu.async_copy(ec2.at[pl.ds(gw, G)],
                         colb.at[pl.ds(slot * G, G)], isem)
        pltpu.async_copy(ev.at[pl.ds(gw * CH, G * CH)],
                         valb.at[pl.ds(slot * G * CH, G * CH)], isem)

    def _stage_wait():
        pltpu.make_async_copy(er2.at[pl.ds(w, G)],
                              rowb.at[pl.ds(0, G)], isem).wait()
        pltpu.make_async_copy(ec2.at[pl.ds(w, G)],
                              colb.at[pl.ds(0, G)], isem).wait()
        pltpu.make_async_copy(ev.at[pl.ds(w * CH, G * CH)],
                              valb.at[pl.ds(0, G * CH)], isem).wait()

    def _loc(slot):
        def body(j, cc):
            for q in range(CH // 16):
                rowb[slot * G + j, pl.ds(q * 16, 16)] = (
                    rowb[slot * G + j, pl.ds(q * 16, 16)] - offv)
            return cc
        lax.fori_loop(0, G, body, 0)

    def _start_gather(slot_chunk, b, par):
        pltpu.async_copy(feat.at[colb.at[slot_chunk]],
                         rbuf.at[pl.ds(b * CH, CH)], gsems[par])

    def _wait_gather(b, par):
        pltpu.make_async_copy(feat.at[colb.at[0]],
                              rbuf.at[pl.ds(b * CH, CH)], gsems[par]).wait()

    def _start_scatter(slot_chunk, b, par):
        pltpu.async_copy(rbuf.at[pl.ds(b * CH, CH)],
                         agg.at[rowb.at[slot_chunk]], ssems[par], add=True)

    def _wait_scatter(par):
        pltpu.make_async_copy(rbuf.at[pl.ds(0, CH)],
                              agg.at[rowb.at[0]], ssems[par]).wait()

    def _scale(slot_chunk, b):
        vbase = slot_chunk * CH
        for gg in range(CH // 16):
            vv = valb[pl.ds(vbase + gg * 16, 16)]
            for r in range(16):
                row = b * CH + gg * 16 + r
                vb = _lane_bcast(vv, r)
                for q in range(D // 16):
                    rbuf[row, pl.ds(q * 16, 16)] = (
                        rbuf[row, pl.ds(q * 16, 16)] * vb)

    _stage_sync(w, 0)
    _loc(0)
    _start_gather(0, 0, 0)
    _start_gather(1, 1, 1)

    def _group(g, carry):
        gslot = lax.rem(g, NSLOT)
        nslot = lax.rem(g + 1, NSLOT)
        for j in range(G):
            b = j
            par = j % 2
            _wait_gather(b, par)
            if j == 0:
                @pl.when(g + 1 < NG)
                def _():
                    _stage_async(w + (g + 1) * G, nslot)
            _scale(gslot * G + j, b)
            if j < 2:
                @pl.when(g > 0)
                def _():
                    _wait_scatter(par)
            else:
                _wait_scatter(par)
            _start_scatter(gslot * G + j, b, par)
            if j < 2:
                _start_gather(gslot * G + j + 2, (j + 2) % NBUF, par)
            else:
                @pl.when(g + 1 < NG)
                def _():
                    if j == 2:
                        _stage_wait()
                        _loc(nslot)
                    _start_gather(nslot * G + (j - 2), (j + 2) % NBUF, par)
        return carry
    lax.fori_loop(0, NG, _group, 0)
    _wait_scatter(0)
    _wait_scatter(1)
    plsc.subcore_barrier()

    last = NS - 1
    @pl.when((c == 0) & (s < last))
    def _():
        pltpu.sync_copy(agg.at[pl.ds(s * RPT0, RPT0)],
                        out.at[pl.ds(s * RPT0, RPT0)])
    @pl.when((c == 0) & (s == last))
    def _():
        pltpu.sync_copy(agg.at[pl.ds(last * RPT0, RPT0_LAST)],
                        out.at[pl.ds(last * RPT0, RPT0_LAST)])
    @pl.when((c == 1) & (s < last))
    def _():
        pltpu.sync_copy(agg.at[pl.ds(s * RPT1, RPT1)],
                        out.at[pl.ds(NU + s * RPT1, RPT1)])
    @pl.when((c == 1) & (s == last))
    def _():
        pltpu.sync_copy(agg.at[pl.ds(last * RPT1, RPT1_LAST)],
                        out.at[pl.ds(NU + last * RPT1, RPT1_LAST)])


_SC_PARAMS = pltpu.CompilerParams(use_tc_tiling_on_sc=False)

_spmm = functools.partial(
    pl.kernel,
    out_type=jax.ShapeDtypeStruct((NNODES, D), jnp.float32),
    mesh=_MESH,
    compiler_params=_SC_PARAMS,
    scratch_types=[
        pltpu.VMEM_SHARED((NU, D), jnp.float32),
        pltpu.SemaphoreType.DMA,
        pltpu.SemaphoreType.DMA,
        pltpu.SemaphoreType.DMA,
        pltpu.SemaphoreType.DMA,
        pltpu.SemaphoreType.DMA,
    ],
)(_spmm_body)


BW = 4096 // (NC * NS)


def _gather_body(f0, f1, f2, f3, users, items,
                 u0, u1, u2, u3, i0, i1, i2, i3, sem):
    pl.run_scoped(
        functools.partial(_gather_inner, f0, f1, f2, f3, users, items,
                          u0, u1, u2, u3, i0, i1, i2, i3, sem),
        idxb=pltpu.VMEM((BW,), jnp.int32),
        rbuf=pltpu.VMEM((BW, D), jnp.float32),
    )


def _gather_inner(f0, f1, f2, f3, users, items,
                  u0, u1, u2, u3, i0, i1, i2, i3, sem, idxb, rbuf):
    c = lax.axis_index("c")
    s = lax.axis_index("s")
    base = (s * NC + c) * BW
    pltpu.sync_copy(users.at[pl.ds(base, BW)], idxb)
    for f, og in ((f0, u0), (f1, u1), (f2, u2), (f3, u3)):
        pltpu.async_copy(f.at[idxb], rbuf, sem).wait()
        pltpu.sync_copy(rbuf, og.at[pl.ds(base, BW)])
    pltpu.sync_copy(items.at[pl.ds(base, BW)], idxb)
    offv = jnp.full((16,), NU, jnp.int32)
    def _off(q, carry):
        idxb[pl.ds(q * 16, 16)] = idxb[pl.ds(q * 16, 16)] + offv
        return carry
    lax.fori_loop(0, BW // 16, _off, 0)
    for f, og in ((f0, i0), (f1, i1), (f2, i2), (f3, i3)):
        pltpu.async_copy(f.at[idxb], rbuf, sem).wait()
        pltpu.sync_copy(rbuf, og.at[pl.ds(base, BW)])


_gather = functools.partial(
    pl.kernel,
    out_type=[jax.ShapeDtypeStruct((4096, D), jnp.float32)] * 8,
    mesh=_MESH,
    compiler_params=_SC_PARAMS,
    scratch_types=[
        pltpu.SemaphoreType.DMA,
    ],
)(_gather_body)


def _dense_layer(agg, feat, w1, w2, bsum):
    M = agg.shape[0]
    blk = 2000

    def body(a_ref, f_ref, w1_ref, w2_ref, b_ref, o_ref):
        a = a_ref[...]
        f = f_ref[...]
        y = jnp.dot(a + f, w1_ref[...], preferred_element_type=jnp.float32)
        y = y + jnp.dot(a * f, w2_ref[...], preferred_element_type=jnp.float32)
        y = y + b_ref[...]
        o_ref[...] = jnp.where(y > 0, y, 0.01 * y)

    return pl.pallas_call(
        body,
        grid=(M // blk,),
        in_specs=[
            pl.BlockSpec((blk, D), lambda i: (i, 0)),
            pl.BlockSpec((blk, D), lambda i: (i, 0)),
            pl.BlockSpec((D, D), lambda i: (0, 0)),
            pl.BlockSpec((D, D), lambda i: (0, 0)),
            pl.BlockSpec((1, D), lambda i: (0, 0)),
        ],
        out_specs=pl.BlockSpec((blk, D), lambda i: (i, 0)),
        out_shape=jax.ShapeDtypeStruct((M, D), jnp.float32),
    )(agg, feat, w1, w2, bsum)


def _predict(us, its, wts, bt):
    B = us[0].shape[0]

    def body(u0, u1, u2, u3, i0, i1, i2, i3, w0, w1, w2, w3, b, o):
        ue = b[...].astype(jnp.float32)
        ie = b[...].astype(jnp.float32)
        for u, i, w in ((u0, i0, w0), (u1, i1, w1), (u2, i2, w2), (u3, i3, w3)):
            ue = ue + jnp.dot(u[...], w[...], preferred_element_type=jnp.float32)
            ie = ie + jnp.dot(i[...], w[...], preferred_element_type=jnp.float32)
        o[...] = jnp.sum(ue * ie, axis=1, keepdims=True)

    return pl.pallas_call(
        body,
        out_shape=jax.ShapeDtypeStruct((B, 1), jnp.float32),
    )(*us, *its, *wts, bt)


def kernel(users, items, features, edge_row, edge_col, edge_val,
           W1, b1, W2, b2, Wt, bt):
    E = edge_row.shape[0]
    EH = E // 2
    npad = EH_PAD - EH

    def _pad_half(x, h, padval):
        seg = lax.slice(x, (h * EH,), ((h + 1) * EH,))
        return jnp.concatenate([seg, jnp.full((npad,), padval, x.dtype)])

    er2 = jnp.concatenate(
        [_pad_half(edge_row, 0, 0), _pad_half(edge_row, 1, NU)]).reshape(-1, CH)
    ec2 = jnp.concatenate(
        [_pad_half(edge_col, 0, 0), _pad_half(edge_col, 1, 0)]).reshape(-1, CH)
    ev = jnp.concatenate(
        [_pad_half(edge_val, 0, 0.0), _pad_half(edge_val, 1, 0.0)])

    feats = [features]
    f = features
    for l in range(NLAYER):
        agg = _spmm(f, er2, ec2, ev)
        f = _dense_layer(agg, f, W1[l], W2[l], (b1[l] + b2[l]).reshape(1, D))
        feats.append(f)

    g = _gather(feats[0], feats[1], feats[2], feats[3], users, items)
    wts = [Wt[k * D:(k + 1) * D, :] for k in range(NLAYER + 1)]
    pred = _predict(g[:4], g[4:], wts, bt.reshape(1, D))
    return pred.reshape(-1)
